# baseline (speedup 1.0000x reference)
import functools

import jax
import jax.numpy as jnp
from jax import lax
from jax.experimental import pallas as pl
from jax.experimental.pallas import tpu as pltpu
from jax.experimental.pallas import tpu_sc as plsc

BATCH = 16384
D = 32
NC = 2
NS = 16
L = 16
NW = NC * NS
BPW = BATCH // NW
CHUNK = 128
NCHUNK = BPW // CHUNK


_mesh = plsc.VectorSubcoreMesh(
    core_axis_name="c", subcore_axis_name="s", num_cores=NC, num_subcores=NS
)


@functools.partial(
    pl.kernel,
    out_type=jax.ShapeDtypeStruct((BATCH,), jnp.float32),
    mesh=_mesh,
    compiler_params=pltpu.CompilerParams(
        needs_layout_passes=False, use_tc_tiling_on_sc=False
    ),
    scratch_types=[
        pltpu.VMEM((NCHUNK, CHUNK), jnp.int32),
        pltpu.VMEM((NCHUNK, CHUNK), jnp.int32),
        pltpu.VMEM((BPW, D), jnp.float32),
        pltpu.VMEM((BPW, D), jnp.float32),
        pltpu.VMEM((BPW,), jnp.float32),
        pltpu.SemaphoreType.DMA,
    ],
)
def _mf_kernel(user_hbm, item_hbm, ut_hbm, it_hbm, out_hbm,
               uidx, iidx, urows, irows, outv, sem):
    wid = lax.axis_index("s") * NC + lax.axis_index("c")
    base = wid * BPW

    pltpu.sync_copy(user_hbm.at[pl.ds(wid * NCHUNK, NCHUNK)], uidx)
    pltpu.sync_copy(item_hbm.at[pl.ds(wid * NCHUNK, NCHUNK)], iidx)

    cps = []
    for j in range(NCHUNK):
        cps.append(pltpu.async_copy(ut_hbm.at[uidx.at[j]],
                                    urows.at[pl.ds(j * CHUNK, CHUNK)], sem))
        cps.append(pltpu.async_copy(it_hbm.at[iidx.at[j]],
                                    irows.at[pl.ds(j * CHUNK, CHUNK)], sem))
    for cp in cps:
        cp.wait()

    lane = lax.iota(jnp.int32, L)

    def body(g, carry):
        rows = g * L + lane
        acc = jnp.zeros((L,), jnp.float32)
        for d in range(D):
            dv = jnp.full((L,), d, jnp.int32)
            acc = acc + (plsc.load_gather(urows, [rows, dv])
                         * plsc.load_gather(irows, [rows, dv]))
        outv[pl.ds(g * L, L)] = acc
        return carry

    lax.fori_loop(0, BPW // L, body, 0)

    pltpu.sync_copy(outv, out_hbm.at[pl.ds(base, BPW)])


def kernel(user, item, user_table, item_table):
    user2d = user.reshape(NW * NCHUNK, CHUNK)
    item2d = item.reshape(NW * NCHUNK, CHUNK)
    return _mf_kernel(user2d, item2d, user_table, item_table)
